# baseline (speedup 1.0000x reference)
import functools

import jax
import jax.numpy as jnp
from jax import lax
from jax.experimental import pallas as pl
from jax.experimental.pallas import tpu as pltpu
from jax.experimental.pallas import tpu_sc as plsc

E = 8
B = 4096
K = 2048
CM = 2048
N = 1024

R = 256
B_PAD = B + E * R
T = B_PAD // R

NC = 2
NS = 16
NW = NC * NS


def _route(idxs_b):
    counts = jnp.bincount(idxs_b, length=E).astype(jnp.int32)
    tiles_per = (counts + R - 1) // R
    padded_start = (jnp.cumsum(tiles_per) - tiles_per) * R
    start_sorted = jnp.cumsum(counts) - counts
    order = jnp.argsort(idxs_b).astype(jnp.int32)
    sorted_ids = idxs_b[order]
    p = jnp.arange(B, dtype=jnp.int32)
    rank = p - start_sorted[sorted_ids]
    dst_pos = (padded_start[sorted_ids] + rank).astype(jnp.int32)
    src = (jnp.arange(B_PAD, dtype=jnp.int32) % B).at[dst_pos].set(order)
    dest = jnp.zeros((B,), jnp.int32).at[order].set(dst_pos)
    tile_start = jnp.arange(T, dtype=jnp.int32) * R
    gid = jnp.sum(tile_start[:, None] >= padded_start[None, :], axis=1) - 1
    gid = jnp.clip(gid, 0, E - 1).astype(jnp.int32)
    return src, dest, gid


def _sc_gather(table, idx, chunk):
    M = idx.shape[0]
    D = table.shape[1]
    m_per_w = M // NW
    n = m_per_w // chunk
    NBUF = 3
    assert n >= NBUF and m_per_w % chunk == 0 and chunk % 8 == 0
    mesh = plsc.VectorSubcoreMesh(core_axis_name="c", subcore_axis_name="s")

    @functools.partial(
        pl.kernel,
        mesh=mesh,
        out_type=jax.ShapeDtypeStruct((M, D), table.dtype),
        scratch_types=[
            pltpu.VMEM((m_per_w,), jnp.int32),
            [pltpu.VMEM((chunk, D), table.dtype)] * NBUF,
            [pltpu.SemaphoreType.DMA] * NBUF,
            [pltpu.SemaphoreType.DMA] * NBUF,
        ],
    )
    def k(table_hbm, idx_hbm, out_hbm, idx_v, bufs, sg, sw):
        wid = lax.axis_index("s") * NC + lax.axis_index("c")
        base = wid * m_per_w
        pltpu.sync_copy(idx_hbm.at[pl.ds(base, m_per_w)], idx_v)

        def gcopy(j):
            b = j % NBUF
            return pltpu.make_async_copy(
                table_hbm.at[idx_v.at[pl.ds(j * chunk, chunk)]], bufs[b], sg[b])

        def wcopy(j):
            b = j % NBUF
            return pltpu.make_async_copy(
                bufs[b], out_hbm.at[pl.ds(base + j * chunk, chunk)], sw[b])

        for j in range(NBUF - 1):
            gcopy(j).start()
        for j in range(n):
            nj = j + NBUF - 1
            if nj < n:
                if nj >= NBUF:
                    wcopy(nj - NBUF).wait()
                gcopy(nj).start()
            gcopy(j).wait()
            wcopy(j).start()
        for j in range(n - NBUF, n):
            wcopy(j).wait()

    return k(table, idx)


def _mlp_body(gid_ref, x_ref, w1_ref, b1_ref, w2_ref, b2_ref, o_ref):
    x = x_ref[...]
    h = jnp.maximum(
        jnp.dot(x, w1_ref[0].astype(jnp.bfloat16),
                preferred_element_type=jnp.float32)
        + b1_ref[0],
        0.0,
    ).astype(jnp.bfloat16)
    o_ref[...] = (
        jnp.dot(h, w2_ref[0].astype(jnp.bfloat16),
                preferred_element_type=jnp.float32)
        + b2_ref[0]
    )


def _mlp(xs, W1, b1, W2, b2, gid):
    grid_spec = pltpu.PrefetchScalarGridSpec(
        num_scalar_prefetch=1,
        grid=(T,),
        in_specs=[
            pl.BlockSpec((R, K), lambda t, g: (t, 0)),
            pl.BlockSpec((1, K, CM), lambda t, g: (g[t], 0, 0)),
            pl.BlockSpec((1, 1, CM), lambda t, g: (g[t], 0, 0)),
            pl.BlockSpec((1, CM, N), lambda t, g: (g[t], 0, 0)),
            pl.BlockSpec((1, 1, N), lambda t, g: (g[t], 0, 0)),
        ],
        out_specs=pl.BlockSpec((R, N), lambda t, g: (t, 0)),
    )
    return pl.pallas_call(
        _mlp_body,
        grid_spec=grid_spec,
        out_shape=jax.ShapeDtypeStruct((B_PAD, N), jnp.float32),
        compiler_params=pltpu.CompilerParams(
            dimension_semantics=("arbitrary",),
            vmem_limit_bytes=100 * 1024 * 1024,
        ),
    )(gid, xs, W1, b1.reshape(E, 1, CM), W2, b2.reshape(E, 1, N))


def kernel(xis, idxs_b, W1, b1, W2, b2):
    src, dest, gid = _route(idxs_b)
    xp = jax.lax.bitcast_convert_type(
        xis.astype(jnp.bfloat16).reshape(B, K // 2, 2), jnp.float32)
    xsp = _sc_gather(xp, src, 32)
    xs = jax.lax.bitcast_convert_type(xsp, jnp.bfloat16).reshape(B_PAD, K)
    ys = _mlp(xs, W1, b1, W2, b2, gid)
    return _sc_gather(ys, dest, 32)


# speedup vs baseline: 1.0533x; 1.0533x over previous
import functools

import jax
import jax.numpy as jnp
from jax import lax
from jax.experimental import pallas as pl
from jax.experimental.pallas import tpu as pltpu
from jax.experimental.pallas import tpu_sc as plsc

E = 8
B = 4096
K = 2048
CM = 2048
N = 1024

R = 256
B_PAD = B + E * R
T = B_PAD // R

NC = 2
NS = 16
NW = NC * NS


def _route(idxs_b):
    counts = jnp.bincount(idxs_b, length=E).astype(jnp.int32)
    tiles_per = (counts + R - 1) // R
    padded_start = (jnp.cumsum(tiles_per) - tiles_per) * R
    start_sorted = jnp.cumsum(counts) - counts
    order = jnp.argsort(idxs_b).astype(jnp.int32)
    sorted_ids = idxs_b[order]
    p = jnp.arange(B, dtype=jnp.int32)
    rank = p - start_sorted[sorted_ids]
    dst_pos = (padded_start[sorted_ids] + rank).astype(jnp.int32)
    src = (jnp.arange(B_PAD, dtype=jnp.int32) % B).at[dst_pos].set(order)
    dest = jnp.zeros((B,), jnp.int32).at[order].set(dst_pos)
    tile_start = jnp.arange(T, dtype=jnp.int32) * R
    gid = jnp.sum(tile_start[:, None] >= padded_start[None, :], axis=1) - 1
    gid = jnp.clip(gid, 0, E - 1).astype(jnp.int32)

    t_arr = jnp.arange(T, dtype=jnp.int32)
    real = (tile_start - padded_start[gid] < counts[gid]).astype(jnp.int32)
    run_first = jnp.concatenate(
        [jnp.ones((1,), jnp.int32), (gid[1:] != gid[:-1]).astype(jnp.int32)])
    ri = jnp.cumsum(run_first) - 1
    nr = ri[-1] + 1
    slot = (ri % 2).astype(jnp.int32)
    rsg = jnp.zeros((T,), jnp.int32).at[ri].set(gid)
    f_g = rsg[jnp.clip(ri + 1, 0, T - 1)]
    f_flag = (run_first == 1) & (ri + 1 < nr)
    f_slot = ((ri + 1) % 2).astype(jnp.int32)
    xtile = jax.lax.cummax(jnp.where(real == 1, t_arr, -1), axis=0)
    meta = jnp.stack([
        gid,
        slot,
        run_first,
        f_flag.astype(jnp.int32),
        f_g,
        f_slot,
        real,
        xtile,
        rsg,
    ])
    return src, dest, meta


def _sc_gather(table, idx, chunk):
    M = idx.shape[0]
    D = table.shape[1]
    m_per_w = M // NW
    n = m_per_w // chunk
    NBUF = 3
    assert n >= NBUF and m_per_w % chunk == 0 and chunk % 8 == 0
    mesh = plsc.VectorSubcoreMesh(core_axis_name="c", subcore_axis_name="s")

    @functools.partial(
        pl.kernel,
        mesh=mesh,
        out_type=jax.ShapeDtypeStruct((M, D), table.dtype),
        scratch_types=[
            pltpu.VMEM((m_per_w,), jnp.int32),
            [pltpu.VMEM((chunk, D), table.dtype)] * NBUF,
            [pltpu.SemaphoreType.DMA] * NBUF,
            [pltpu.SemaphoreType.DMA] * NBUF,
        ],
    )
    def k(table_hbm, idx_hbm, out_hbm, idx_v, bufs, sg, sw):
        wid = lax.axis_index("s") * NC + lax.axis_index("c")
        base = wid * m_per_w
        pltpu.sync_copy(idx_hbm.at[pl.ds(base, m_per_w)], idx_v)

        def gcopy(j):
            b = j % NBUF
            return pltpu.make_async_copy(
                table_hbm.at[idx_v.at[pl.ds(j * chunk, chunk)]], bufs[b], sg[b])

        def wcopy(j):
            b = j % NBUF
            return pltpu.make_async_copy(
                bufs[b], out_hbm.at[pl.ds(base + j * chunk, chunk)], sw[b])

        for j in range(NBUF - 1):
            gcopy(j).start()
        for j in range(n):
            nj = j + NBUF - 1
            if nj < n:
                if nj >= NBUF:
                    wcopy(nj - NBUF).wait()
                gcopy(nj).start()
            gcopy(j).wait()
            wcopy(j).start()
        for j in range(n - NBUF, n):
            wcopy(j).wait()

    return k(table, idx)


def _mlp_body(m_ref, x_ref, w1_hbm, b1_ref, w2_hbm, b2_ref, o_ref,
              w1b, w2b, s1, s2):
    t = pl.program_id(0)

    def start_fetch(g, sl):
        pltpu.make_async_copy(w1_hbm.at[g], w1b.at[sl], s1.at[sl]).start()
        pltpu.make_async_copy(w2_hbm.at[g], w2b.at[sl], s2.at[sl]).start()

    @pl.when(t == 0)
    def _():
        start_fetch(m_ref[8, 0], 0)

    @pl.when((m_ref[2, t] == 1) & (m_ref[3, t] == 1))
    def _():
        start_fetch(m_ref[4, t], m_ref[5, t])

    @pl.when(m_ref[2, t] == 1)
    def _():
        g = m_ref[0, t]
        sl = m_ref[1, t]
        pltpu.make_async_copy(w1_hbm.at[g], w1b.at[sl], s1.at[sl]).wait()
        pltpu.make_async_copy(w2_hbm.at[g], w2b.at[sl], s2.at[sl]).wait()

    @pl.when(m_ref[6, t] == 1)
    def _():
        sl = m_ref[1, t]
        x = x_ref[...]
        h = jnp.maximum(
            jnp.dot(x, w1b[sl].astype(jnp.bfloat16),
                    preferred_element_type=jnp.float32)
            + b1_ref[0],
            0.0,
        ).astype(jnp.bfloat16)
        o_ref[...] = (
            jnp.dot(h, w2b[sl].astype(jnp.bfloat16),
                    preferred_element_type=jnp.float32)
            + b2_ref[0]
        )


def _mlp(xs, W1, b1, W2, b2, meta):
    grid_spec = pltpu.PrefetchScalarGridSpec(
        num_scalar_prefetch=1,
        grid=(T,),
        in_specs=[
            pl.BlockSpec((R, K), lambda t, m: (m[7, t], 0)),
            pl.BlockSpec(memory_space=pl.ANY),
            pl.BlockSpec((1, 1, CM), lambda t, m: (m[0, t], 0, 0)),
            pl.BlockSpec(memory_space=pl.ANY),
            pl.BlockSpec((1, 1, N), lambda t, m: (m[0, t], 0, 0)),
        ],
        out_specs=pl.BlockSpec((R, N), lambda t, m: (t, 0)),
        scratch_shapes=[
            pltpu.VMEM((2, K, CM), jnp.float32),
            pltpu.VMEM((2, CM, N), jnp.float32),
            pltpu.SemaphoreType.DMA((2,)),
            pltpu.SemaphoreType.DMA((2,)),
        ],
    )
    return pl.pallas_call(
        _mlp_body,
        grid_spec=grid_spec,
        out_shape=jax.ShapeDtypeStruct((B_PAD, N), jnp.float32),
        compiler_params=pltpu.CompilerParams(
            dimension_semantics=("arbitrary",),
            vmem_limit_bytes=110 * 1024 * 1024,
        ),
    )(meta, xs, W1, b1.reshape(E, 1, CM), W2, b2.reshape(E, 1, N))


def kernel(xis, idxs_b, W1, b1, W2, b2):
    src, dest, meta = _route(idxs_b)
    xp = jax.lax.bitcast_convert_type(
        xis.astype(jnp.bfloat16).reshape(B, K // 2, 2), jnp.float32)
    xsp = _sc_gather(xp, src, 32)
    xs = jax.lax.bitcast_convert_type(xsp, jnp.bfloat16).reshape(B_PAD, K)
    ys = _mlp(xs, W1, b1, W2, b2, meta)
    return _sc_gather(ys, dest, 32)
